# baseline (speedup 1.0000x reference)
import functools

import jax
import jax.numpy as jnp
from jax import lax
from jax.experimental import pallas as pl
from jax.experimental.pallas import tpu as pltpu
from jax.experimental.pallas import tpu_sc as plsc

B = 16
C = 256
N = 4096
K = 512
L = 16
RADIX = 32
DIGIT_BITS = 5
NUM_PASSES = 7
SCAN_BASE = 1
MIN32 = -2147483648


def _thresh(keys2d):
    u = jax.lax.bitwise_xor(keys2d, jnp.int32(MIN32))
    rows = keys2d.shape[0]
    prefix = jnp.zeros((rows, 1), jnp.int32)
    for j in range(31, -1, -1):
        low = jnp.int32((1 << j) - 1)
        bit = jnp.int32(MIN32) if j == 31 else jnp.int32(1 << j)
        trial = (prefix | low) ^ jnp.int32(MIN32)
        cnt = jnp.sum((u <= trial).astype(jnp.int32), axis=1, keepdims=True)
        prefix = jnp.where(cnt >= K, prefix, prefix | bit)
    n_lt = jnp.sum((u < (prefix ^ jnp.int32(MIN32))).astype(jnp.int32),
                   axis=1, keepdims=True)
    return prefix, n_lt


def _norm_body(src_ref, tgt_ref, keys_ref, meta_ref, embt_ref, keys_acc):
    b = pl.program_id(0)
    x = src_ref[0]
    nx = jnp.sqrt(jnp.sum(x * x, axis=0))
    kx = jnp.bitwise_not(lax.bitcast_convert_type(nx, jnp.int32))
    embt_ref[0, 0] = x.T
    y = tgt_ref[0]
    ny = jnp.sqrt(jnp.sum(y * y, axis=0))
    ky = jnp.bitwise_not(lax.bitcast_convert_type(ny, jnp.int32))
    embt_ref[1, 0] = y.T
    kk = jnp.stack([kx, ky])
    keys_ref[0] = kk
    keys_acc[pl.ds(b, 1)] = kk[None]

    @pl.when(b == B - 1)
    def _():
        t, n_lt = _thresh(keys_acc[...].reshape(2 * B, N))
        ii = lax.broadcasted_iota(jnp.int32, (2 * B, 128), 1)
        meta = jnp.where(ii == 0, t, jnp.where(ii == 1, n_lt, 0))
        meta_ref[...] = meta.reshape(B, 2, 128)


_norms_call = pl.pallas_call(
    _norm_body,
    grid=(B,),
    in_specs=[
        pl.BlockSpec((1, C, N), lambda b: (b, 0, 0)),
        pl.BlockSpec((1, C, N), lambda b: (b, 0, 0)),
    ],
    out_specs=[
        pl.BlockSpec((1, 2, N), lambda b: (b, 0, 0)),
        pl.BlockSpec((B, 2, 128), lambda b: (0, 0, 0)),
        pl.BlockSpec((2, 1, N, C), lambda b: (0, b, 0, 0)),
    ],
    out_shape=[
        jax.ShapeDtypeStruct((B, 2, N), jnp.int32),
        jax.ShapeDtypeStruct((B, 2, 128), jnp.int32),
        jax.ShapeDtypeStruct((2, B, N, C), jnp.float32),
    ],
    scratch_shapes=[pltpu.VMEM((B, 2, N), jnp.int32)],
)


def _digit(k, shift):
    if shift:
        k = lax.shift_right_logical(k, jnp.full((L,), shift, jnp.int32))
    return jnp.bitwise_and(k, RADIX - 1)


_SC_MESH = plsc.VectorSubcoreMesh(core_axis_name="c", subcore_axis_name="s")
_SC_PARAMS = pltpu.CompilerParams(needs_layout_passes=False)

_CAND = K + 2 * L
_TIES = N + L
_CHUNK = 128


@functools.partial(
    pl.kernel,
    out_type=[
        jax.ShapeDtypeStruct((2, B, 3, K), jnp.float32),
        jax.ShapeDtypeStruct((2, B, K, C), jnp.float32),
    ],
    mesh=_SC_MESH,
    compiler_params=_SC_PARAMS,
    scratch_types=[
        pltpu.VMEM((N,), jnp.int32),
        pltpu.VMEM((128,), jnp.int32),
        pltpu.VMEM((_CAND,), jnp.int32),
        pltpu.VMEM((_CAND,), jnp.int32),
        pltpu.VMEM((_TIES,), jnp.int32),
        pltpu.VMEM((K,), jnp.int32),
        pltpu.VMEM((K,), jnp.int32),
        pltpu.VMEM((K,), jnp.int32),
        pltpu.VMEM((K,), jnp.int32),
        pltpu.VMEM((RADIX,), jnp.int32),
        pltpu.VMEM((6, N), jnp.float32),
        pltpu.VMEM((3, K), jnp.float32),
        pltpu.VMEM((K,), jnp.int32),
        pltpu.VMEM((_CHUNK, C), jnp.float32),
        pltpu.VMEM((_CHUNK, C), jnp.float32),
        pltpu.SemaphoreType.DMA,
        pltpu.SemaphoreType.DMA,
    ],
)
def _sc_topk(keys_ba, meta_ba, src, tgt, embt, kp_out, gath_out,
             keys0, meta_v, cand_k, cand_i, ties_i,
             comb_k0, comb_k1, comb_i0, comb_i1, hist,
             kp_stage, kp_buf, idx_abs, buf_a, buf_b, sem_a, sem_b):
    c = lax.axis_index("c")
    s = lax.axis_index("s")
    lanes = lax.iota(jnp.int32, L)
    minv = jnp.full((L,), MIN32, jnp.int32)

    pltpu.sync_copy(keys_ba.at[s, c], keys0)
    pltpu.sync_copy(meta_ba.at[s, c], meta_v)
    mv = meta_v[pl.ds(0, L)]
    t_key = mv[0]
    n_lt = mv[1]
    tv = jnp.full((L,), 0, jnp.int32) + t_key
    txv = tv ^ minv
    nltv = jnp.full((L,), 0, jnp.int32) + n_lt

    def compact_body(i, carry):
        off_lt, off_eq = carry
        kv = keys0[pl.ds(i * L, L)]
        iv = lanes + i * L
        mlt = (kv ^ minv) < txv
        meq = kv == tv
        plsc.store_compressed(cand_k.at[pl.ds(off_lt, L)], kv, mask=mlt)
        plsc.store_compressed(cand_i.at[pl.ds(off_lt, L)], iv, mask=mlt)
        plsc.store_compressed(ties_i.at[pl.ds(off_eq, L)], iv, mask=meq)
        off_lt = off_lt + jnp.max(plsc.all_reduce_population_count(mlt))
        off_eq = off_eq + jnp.max(plsc.all_reduce_population_count(meq))
        return off_lt, off_eq

    lax.fori_loop(0, N // L, compact_body, (jnp.int32(0), jnp.int32(0)))

    for j in range(K // L):
        pos = lanes + j * L
        m = pos < nltv
        a_k = plsc.load_gather(cand_k, [pos])
        a_i = plsc.load_gather(cand_i, [pos])
        t_i = plsc.load_gather(ties_i, [jnp.maximum(pos - nltv, 0)])
        comb_k0[pl.ds(j * L, L)] = jnp.where(m, a_k, tv)
        comb_i0[pl.ds(j * L, L)] = jnp.where(m, a_i, t_i)

    nvec = K // L

    def zero_hist():
        z = jnp.zeros((L,), jnp.int32)
        hist[pl.ds(0, L)] = z
        hist[pl.ds(L, L)] = z

    def spread_offsets():
        h0 = hist[pl.ds(0, L)]
        h1 = hist[pl.ds(L, L)]
        c0 = plsc.cumsum(h0)
        c1 = plsc.cumsum(h1)
        t0 = jnp.sum(h0)
        hist[pl.ds(0, L)] = c0 - h0
        hist[pl.ds(L, L)] = c1 - h1 + t0

    def hist_add(d, cnt, last):
        plsc.addupdate_scatter(hist, [d], cnt + (1 - SCAN_BASE), mask=last)

    for p in range(NUM_PASSES):
        shift = p * DIGIT_BITS
        kin, vin, kout, vout = (
            (comb_k1, comb_i1, comb_k0, comb_i0) if p % 2
            else (comb_k0, comb_i0, comb_k1, comb_i1)
        )
        zero_hist()

        def p_count(i, carry, kin=kin, shift=shift):
            d = _digit(kin[pl.ds(i * L, L)], shift)
            cnt, last = plsc.scan_count(d)
            hist_add(d, cnt, last)
            return carry

        lax.fori_loop(0, nvec, p_count, 0)
        spread_offsets()

        def p_perm(i, carry, kin=kin, vin=vin, kout=kout, vout=vout, shift=shift):
            key = kin[pl.ds(i * L, L)]
            val = vin[pl.ds(i * L, L)]
            d = _digit(key, shift)
            cnt, last = plsc.scan_count(d)
            base = plsc.load_gather(hist, [d])
            pos = base + cnt - SCAN_BASE
            plsc.store_scatter(kout, [pos], key)
            plsc.store_scatter(vout, [pos], val)
            hist_add(d, cnt, last)
            return carry

        lax.fori_loop(0, nvec, p_perm, 0)

    sorted_vals = comb_i1 if NUM_PASSES % 2 else comb_i0

    pltpu.sync_copy(src.at[s], kp_stage.at[pl.ds(0, 3)])
    pltpu.sync_copy(tgt.at[s], kp_stage.at[pl.ds(3, 3)])
    for v in range(K // L):
        iv = sorted_vals[pl.ds(v * L, L)]
        for ch in range(3):
            chv = jnp.full((L,), ch, jnp.int32) + c * 3
            g = plsc.load_gather(kp_stage, [chv, iv])
            kp_buf[ch, pl.ds(v * L, L)] = g
    pltpu.sync_copy(kp_buf, kp_out.at[c, s])

    base = (c * B + s) * N
    for v in range(K // L):
        idx_abs[pl.ds(v * L, L)] = sorted_vals[pl.ds(v * L, L)] + base

    def start(chunk, buf, sem):
        pltpu.async_copy(embt.at[idx_abs.at[pl.ds(chunk * _CHUNK, _CHUNK)]],
                         buf, sem)

    def finish(chunk, buf, sem):
        pltpu.make_async_copy(
            embt.at[idx_abs.at[pl.ds(chunk * _CHUNK, _CHUNK)]],
            buf, sem).wait()
        pltpu.sync_copy(buf, gath_out.at[c, s, pl.ds(chunk * _CHUNK, _CHUNK), :])

    start(0, buf_a, sem_a)
    start(1, buf_b, sem_b)
    finish(0, buf_a, sem_a)
    start(2, buf_a, sem_a)
    finish(1, buf_b, sem_b)
    start(3, buf_b, sem_b)
    finish(2, buf_a, sem_a)
    finish(3, buf_b, sem_b)


def _tpose_body(g_ref, o_ref):
    o_ref[0] = g_ref[0, 0].T


def _tpose_call(side):
    return pl.pallas_call(
        _tpose_body,
        grid=(B,),
        in_specs=[pl.BlockSpec((1, 1, K, C), lambda b: (side, b, 0, 0))],
        out_specs=pl.BlockSpec((1, C, K), lambda b: (b, 0, 0)),
        out_shape=jax.ShapeDtypeStruct((B, C, K), jnp.float32),
    )


def kernel(src, tgt, src_embedding, tgt_embedding):
    keys_ba, meta_ba, embt = _norms_call(src_embedding, tgt_embedding)
    kp_out, gath = _sc_topk(keys_ba, meta_ba, src, tgt,
                            embt.reshape(2 * B * N, C))
    semb_out = _tpose_call(0)(gath)
    temb_out = _tpose_call(1)(gath)
    return kp_out[0], kp_out[1], semb_out, temb_out
